# baseline (speedup 1.0000x reference)
import jax
import jax.numpy as jnp
from jax import lax
from jax.experimental import pallas as pl
from jax.experimental.pallas import tpu as pltpu
from jax.experimental.pallas import tpu_sc as plsc

_ATOM_WEIGHTS = [1.008, 12.011, 14.007, 15.999, 18.998, 30.974, 32.06,
                 35.453, 79.904, 126.904, 10.811, 28.086, 78.971, 22.99,
                 39.098, 6.941]
_MAX_WEIGHT = 126.904
_WNORM = [w / _MAX_WEIGHT for w in _ATOM_WEIGHTS]

_B = 16384
_A = 200
_C = 16
_NW = 32
_B_SC = 8192
_B_TC = _B - _B_SC
_MOLS_PER_W = _B_SC // _NW
_KA = 4
_NCH = _A // _KA
_NG = _MOLS_PER_W // _C
_TC_BM = 1024


def _argmax_weight(vals, weights):
  items = list(zip(vals, weights))
  while len(items) > 1:
    nxt = []
    for i in range(0, len(items), 2):
      v1, w1 = items[i]
      v2, w2 = items[i + 1]
      gt = v2 > v1
      nxt.append((jnp.where(gt, v2, v1), jnp.where(gt, w2, w1)))
    items = nxt
  return items[0][1]


def _tec_body(y_hbm, out_hbm, buf0, buf1, acc_v, sem0, sem1):
  wid = lax.axis_index("s") * 2 + lax.axis_index("c")
  mol0 = wid * _MOLS_PER_W

  zero = jnp.zeros((16,), jnp.float32)
  wsplats = [jnp.full((16,), w, jnp.float32) for w in _WNORM]
  bufs = (buf0, buf1)
  sems = (sem0, sem1)

  def src(ci):
    return y_hbm.at[pl.ds(ci * _KA, _KA), :, pl.ds(mol0, _MOLS_PER_W)]

  def init_body(g, carry):
    acc_v[pl.ds(g * _C, _C)] = zero
    return carry

  lax.fori_loop(0, _NG, init_body, 0)

  def compute(buf):
    def group_body(g, c2):
      m0 = g * _C
      acc = acc_v[pl.ds(m0, _C)]
      for ai in range(_KA):
        vals = [buf[ai, c, pl.ds(m0, _C)] for c in range(_C)]
        acc = acc + _argmax_weight(vals, wsplats)
      acc_v[pl.ds(m0, _C)] = acc
      return c2

    lax.fori_loop(0, _NG, group_body, 0)

  pltpu.async_copy(src(0), buf0, sem0)

  def pair_body(c2, carry):
    ci = 2 * c2
    for b in range(2):
      pltpu.make_async_copy(src(ci + b), bufs[b], sems[b]).wait()
      nxt = ci + b + 1

      @pl.when(nxt < _NCH)
      def _():
        pltpu.async_copy(src(nxt), bufs[1 - b], sems[1 - b])

      compute(bufs[b])
    return carry

  lax.fori_loop(0, _NCH // 2, pair_body, 0)

  pltpu.sync_copy(acc_v, out_hbm.at[pl.ds(mol0, _MOLS_PER_W)])


def _tc_kernel_body(x_ref, o_ref):
  w_lo = jnp.concatenate(
      [jnp.full((1, 128), w, jnp.float32) for w in _WNORM[:8]], axis=0)
  w_hi = jnp.concatenate(
      [jnp.full((1, 128), w, jnp.float32) for w in _WNORM[8:]], axis=0)
  row0 = lax.broadcasted_iota(jnp.int32, (8, 128), 0) == 0
  zero = jnp.zeros((8, 128), jnp.float32)

  def reduce_rows(v, w):
    for s in (1, 2, 4):
      vs = pltpu.roll(v, 8 - s, 0)
      ws = pltpu.roll(w, 8 - s, 0)
      gt = vs > v
      v = jnp.where(gt, vs, v)
      w = jnp.where(gt, ws, w)
    return v, w

  for cb in range(_TC_BM // 128):
    def atom_body(a, acc, cb=cb):
      v0 = x_ref[a, pl.ds(0, 8), pl.ds(cb * 128, 128)]
      v1 = x_ref[a, pl.ds(8, 8), pl.ds(cb * 128, 128)]
      v0, w0 = reduce_rows(v0, w_lo)
      v1, w1 = reduce_rows(v1, w_hi)
      gt = v1 > v0
      w = jnp.where(gt, w1, w0)
      return acc + jnp.where(row0, w, zero)

    acc = lax.fori_loop(0, _A, atom_body, zero)
    o_ref[pl.ds(cb, 1), :] = acc[0:1, :]


_mesh = plsc.VectorSubcoreMesh(core_axis_name="c", subcore_axis_name="s")


@jax.jit
def _weight_feature(x):
  y = jnp.transpose(x, (1, 2, 0))
  out_sc = pl.kernel(
      _tec_body,
      out_type=jax.ShapeDtypeStruct((_B_SC,), jnp.float32),
      mesh=_mesh,
      scratch_types=[
          pltpu.VMEM((_KA, _C, _MOLS_PER_W), jnp.float32),
          pltpu.VMEM((_KA, _C, _MOLS_PER_W), jnp.float32),
          pltpu.VMEM((_MOLS_PER_W,), jnp.float32),
          pltpu.SemaphoreType.DMA,
          pltpu.SemaphoreType.DMA,
      ],
      compiler_params=pltpu.CompilerParams(
          needs_layout_passes=False, use_tc_tiling_on_sc=True),
  )(y)
  out_tc = pl.pallas_call(
      _tc_kernel_body,
      grid=(_B_TC // _TC_BM,),
      in_specs=[pl.BlockSpec((_A, _C, _TC_BM),
                             lambda g: (0, 0, _B_SC // _TC_BM + g))],
      out_specs=pl.BlockSpec((_TC_BM // 128, 128), lambda g: (g, 0)),
      out_shape=jax.ShapeDtypeStruct((_B_TC // 128, 128), jnp.float32),
      compiler_params=pltpu.CompilerParams(
          dimension_semantics=("arbitrary",)),
  )(y)
  return jnp.concatenate([out_sc, out_tc.reshape(_B_TC)])


def kernel(X):
  return _weight_feature(X).reshape(_B, 1)


# speedup vs baseline: 1.7231x; 1.7231x over previous
import jax
import jax.numpy as jnp
from jax import lax
from jax.experimental import pallas as pl
from jax.experimental.pallas import tpu as pltpu
from jax.experimental.pallas import tpu_sc as plsc

_ATOM_WEIGHTS = [1.008, 12.011, 14.007, 15.999, 18.998, 30.974, 32.06,
                 35.453, 79.904, 126.904, 10.811, 28.086, 78.971, 22.99,
                 39.098, 6.941]
_MAX_WEIGHT = 126.904
_WNORM = [w / _MAX_WEIGHT for w in _ATOM_WEIGHTS]

_B = 16384
_A = 200
_C = 16
_NW = 32
_B_SC = 8192
_B_TC = _B - _B_SC
_MOLS_PER_W = _B_SC // _NW
_KA = 4
_NCH = _A // _KA
_NG = _MOLS_PER_W // _C
_TC_BM = 1024


def _argmax_weight(vals, weights):
  items = list(zip(vals, weights))
  while len(items) > 1:
    nxt = []
    for i in range(0, len(items), 2):
      v1, w1 = items[i]
      v2, w2 = items[i + 1]
      gt = v2 > v1
      nxt.append((jnp.where(gt, v2, v1), jnp.where(gt, w2, w1)))
    items = nxt
  return items[0][1]


def _tec_body(y_hbm, out_hbm, buf0, buf1, acc_v, sem0, sem1):
  wid = lax.axis_index("s") * 2 + lax.axis_index("c")
  mol0 = wid * _MOLS_PER_W

  zero = jnp.zeros((16,), jnp.float32)
  wsplats = [jnp.full((16,), w, jnp.float32) for w in _WNORM]
  bufs = (buf0, buf1)
  sems = (sem0, sem1)

  def src(ci):
    return y_hbm.at[pl.ds(ci * _KA, _KA), :, pl.ds(mol0, _MOLS_PER_W)]

  def init_body(g, carry):
    acc_v[pl.ds(g * _C, _C)] = zero
    return carry

  lax.fori_loop(0, _NG, init_body, 0)

  def compute(buf):
    def group_body(g, c2):
      m0 = g * _C
      acc = acc_v[pl.ds(m0, _C)]
      for ai in range(_KA):
        vals = [buf[ai, c, pl.ds(m0, _C)] for c in range(_C)]
        acc = acc + _argmax_weight(vals, wsplats)
      acc_v[pl.ds(m0, _C)] = acc
      return c2

    lax.fori_loop(0, _NG, group_body, 0)

  pltpu.async_copy(src(0), buf0, sem0)

  def pair_body(c2, carry):
    ci = 2 * c2
    for b in range(2):
      pltpu.make_async_copy(src(ci + b), bufs[b], sems[b]).wait()
      nxt = ci + b + 1

      @pl.when(nxt < _NCH)
      def _():
        pltpu.async_copy(src(nxt), bufs[1 - b], sems[1 - b])

      compute(bufs[b])
    return carry

  lax.fori_loop(0, _NCH // 2, pair_body, 0)

  pltpu.sync_copy(acc_v, out_hbm.at[pl.ds(mol0, _MOLS_PER_W)])


def _tc_kernel_body(x_ref, o_ref):
  w_lo = jnp.concatenate(
      [jnp.full((1, 128), w, jnp.float32) for w in _WNORM[:8]], axis=0)
  w_hi = jnp.concatenate(
      [jnp.full((1, 128), w, jnp.float32) for w in _WNORM[8:]], axis=0)
  row0 = lax.broadcasted_iota(jnp.int32, (8, 128), 0) == 0
  zero = jnp.zeros((8, 128), jnp.float32)

  def reduce_rows(v, w):
    for s in (1, 2, 4):
      vs = pltpu.roll(v, 8 - s, 0)
      ws = pltpu.roll(w, 8 - s, 0)
      gt = vs > v
      v = jnp.where(gt, vs, v)
      w = jnp.where(gt, ws, w)
    return v, w

  ncb = _TC_BM // 128

  def atom_body(a, accs):
    out = []
    for cb in range(ncb):
      v0 = x_ref[a, pl.ds(0, 8), pl.ds(cb * 128, 128)]
      v1 = x_ref[a, pl.ds(8, 8), pl.ds(cb * 128, 128)]
      v0, w0 = reduce_rows(v0, w_lo)
      v1, w1 = reduce_rows(v1, w_hi)
      gt = v1 > v0
      w = jnp.where(gt, w1, w0)
      out.append(accs[cb] + jnp.where(row0, w, zero))
    return tuple(out)

  accs = lax.fori_loop(0, _A, atom_body, (zero,) * ncb)
  for cb in range(ncb):
    o_ref[pl.ds(cb, 1), :] = accs[cb][0:1, :]


_mesh = plsc.VectorSubcoreMesh(core_axis_name="c", subcore_axis_name="s")


@jax.jit
def _weight_feature(x):
  y = jnp.transpose(x, (1, 2, 0))
  out_sc = pl.kernel(
      _tec_body,
      out_type=jax.ShapeDtypeStruct((_B_SC,), jnp.float32),
      mesh=_mesh,
      scratch_types=[
          pltpu.VMEM((_KA, _C, _MOLS_PER_W), jnp.float32),
          pltpu.VMEM((_KA, _C, _MOLS_PER_W), jnp.float32),
          pltpu.VMEM((_MOLS_PER_W,), jnp.float32),
          pltpu.SemaphoreType.DMA,
          pltpu.SemaphoreType.DMA,
      ],
      compiler_params=pltpu.CompilerParams(
          needs_layout_passes=False, use_tc_tiling_on_sc=True),
  )(y)
  out_tc = pl.pallas_call(
      _tc_kernel_body,
      grid=(_B_TC // _TC_BM,),
      in_specs=[pl.BlockSpec((_A, _C, _TC_BM),
                             lambda g: (0, 0, _B_SC // _TC_BM + g))],
      out_specs=pl.BlockSpec((_TC_BM // 128, 128), lambda g: (g, 0)),
      out_shape=jax.ShapeDtypeStruct((_B_TC // 128, 128), jnp.float32),
      compiler_params=pltpu.CompilerParams(
          dimension_semantics=("arbitrary",)),
  )(y)
  return jnp.concatenate([out_sc, out_tc.reshape(_B_TC)])


def kernel(X):
  return _weight_feature(X).reshape(_B, 1)
